# baseline (speedup 1.0000x reference)
import functools

import jax
import jax.numpy as jnp
import numpy as np
from jax import lax
from jax.experimental import pallas as pl
from jax.experimental.pallas import tpu as pltpu
from jax.experimental.pallas import tpu_sc as plsc


def _build_sc_embed_sum(B, L, D):
    info = plsc.get_sparse_core_info()
    NC, NS, LANES = info.num_cores, info.num_subcores, info.num_lanes
    NW = NC * NS
    b_per_w = B // NW
    R = 128
    n_chunks = b_per_w // R
    n_vregs = D // LANES
    L_A = min(L, 128)
    L_B = L - L_A

    mesh = plsc.VectorSubcoreMesh(core_axis_name="c", subcore_axis_name="s")

    @functools.partial(
        pl.kernel,
        mesh=mesh,
        compiler_params=pltpu.CompilerParams(
            use_tc_tiling_on_sc=False, needs_layout_passes=False),
        out_type=jax.ShapeDtypeStruct((B, D), jnp.float32),
        scratch_types=[
            pltpu.VMEM((R, L), jnp.int32),
            pltpu.VMEM((4, L, D), jnp.bfloat16),
            pltpu.VMEM((R, D), jnp.float32),
            pltpu.SemaphoreType.DMA,
            pltpu.SemaphoreType.DMA,
            pltpu.SemaphoreType.DMA,
            pltpu.SemaphoreType.DMA,
        ],
    )
    def sc_embed_sum(x_hbm, table_hbm, out_hbm, idxc, rows, outc,
                     sem0, sem1, sem2, sem3):
        wid = lax.axis_index("s") * NC + lax.axis_index("c")
        base = wid * b_per_w
        zero = jnp.zeros((LANES,), jnp.float32)
        sems = (sem0, sem1, sem2, sem3)

        def issue(r, buf, sem):
            pltpu.async_copy(
                table_hbm.at[idxc.at[r, pl.ds(0, L_A)]],
                rows.at[buf, pl.ds(0, L_A)], sem)
            pltpu.async_copy(
                table_hbm.at[idxc.at[r, pl.ds(L_A, L_B)]],
                rows.at[buf, pl.ds(L_A, L_B)], sem)

        def wait(buf, sem):
            pltpu.make_async_copy(
                table_hbm.at[pl.ds(0, L)], rows.at[buf], sem).wait()

        def reduce_into(r, buf):
            def red(j10, accs):
                accs = list(accs)
                for u in range(10):
                    j = j10 * 20 + u * 2
                    for k2 in range(n_vregs // 2):
                        h0 = rows[buf, j, pl.ds(k2 * 2 * LANES, 2 * LANES)]
                        h1 = rows[buf, j + 1, pl.ds(k2 * 2 * LANES, 2 * LANES)]
                        a, b = plsc.unpack(h0 + h1,
                                           format=plsc.PackFormat.INTERLEAVED)
                        accs[2 * k2] = accs[2 * k2] + a
                        accs[2 * k2 + 1] = accs[2 * k2 + 1] + b
                return tuple(accs)

            accs = lax.fori_loop(0, L // 20, red, (zero,) * n_vregs)
            for k in range(n_vregs):
                outc[r, pl.ds(k * LANES, LANES)] = accs[k]

        def chunk_body(c, carry):
            row0 = base + c * R
            pltpu.sync_copy(x_hbm.at[pl.ds(row0, R)], idxc)
            for u in range(4):
                issue(u, u, sems[u])

            def quad_body(r4, carry2):
                r0 = 4 * r4
                for u in range(4):
                    r = r0 + u
                    wait(u, sems[u])
                    reduce_into(r, u)

                    @pl.when(r + 4 < R)
                    def _():
                        issue(r + 4, u, sems[u])
                return carry2

            lax.fori_loop(0, R // 4, quad_body, 0)
            pltpu.sync_copy(outc, out_hbm.at[pl.ds(row0, R)])
            return carry

        lax.fori_loop(0, n_chunks, chunk_body, 0)

    return sc_embed_sum


def _mlp_block(e_ref, w1_ref, b1_ref, w2t_ref, b2_ref, o_ref, *, inv_l):
    e = e_ref[...] * inv_l
    h = jnp.dot(e, w1_ref[...], preferred_element_type=jnp.float32) + b1_ref[...]
    h = jnp.maximum(h, 0.0)
    o_ref[...] = jnp.sum(h * w2t_ref[...], axis=1, keepdims=True) + b2_ref[...]


def _tc_mlp(esum, W1, b1, W2, b2, L):
    B, D = esum.shape
    H = W1.shape[1]
    BLK = 2048
    return pl.pallas_call(
        functools.partial(_mlp_block, inv_l=1.0 / L),
        grid=(B // BLK,),
        in_specs=[
            pl.BlockSpec((BLK, D), lambda i: (i, 0)),
            pl.BlockSpec((D, H), lambda i: (0, 0)),
            pl.BlockSpec((1, H), lambda i: (0, 0)),
            pl.BlockSpec((1, H), lambda i: (0, 0)),
            pl.BlockSpec((1, 1), lambda i: (0, 0)),
        ],
        out_specs=pl.BlockSpec((BLK, 1), lambda i: (i, 0)),
        out_shape=jax.ShapeDtypeStruct((B, 1), jnp.float32),
    )(esum, W1, b1.reshape(1, H), W2.reshape(1, H), b2.reshape(1, 1))


def kernel(x, table, W1, b1, W2, b2):
    B, L = x.shape
    D = table.shape[1]
    esum = _build_sc_embed_sum(B, L, D)(x, table.astype(jnp.bfloat16))
    perm = np.concatenate(
        [np.concatenate([np.arange(g, g + 32, 2), np.arange(g + 1, g + 32, 2)])
         for g in range(0, D, 32)])
    out = _tc_mlp(esum, W1[perm, :], b1, W2, b2, L)
    return out[:, 0]


# speedup vs baseline: 1.3378x; 1.3378x over previous
import functools

import jax
import jax.numpy as jnp
import numpy as np
from jax import lax
from jax.experimental import pallas as pl
from jax.experimental.pallas import tpu as pltpu
from jax.experimental.pallas import tpu_sc as plsc


def _build_sc_embed_sum(B, L, D, V):
    info = plsc.get_sparse_core_info()
    NC, NS, LANES = info.num_cores, info.num_subcores, info.num_lanes
    NW = NC * NS
    b_per_w = B // NW
    R = 128
    n_chunks = b_per_w // R
    n_vregs = D // LANES
    L_A = min(L, 128)
    L_B = L - L_A

    mesh = plsc.VectorSubcoreMesh(core_axis_name="c", subcore_axis_name="s")

    @functools.partial(
        pl.kernel,
        mesh=mesh,
        compiler_params=pltpu.CompilerParams(
            use_tc_tiling_on_sc=False, needs_layout_passes=False),
        out_type=jax.ShapeDtypeStruct((B, D), jnp.float32),
        scratch_types=[
            pltpu.VMEM((R, L), jnp.int32),
            pltpu.VMEM((4, L, D), jnp.bfloat16),
            pltpu.VMEM((R, D), jnp.float32),
            pltpu.VMEM_SHARED((V, D), jnp.bfloat16),
            pltpu.SemaphoreType.DMA,
            pltpu.SemaphoreType.DMA,
            pltpu.SemaphoreType.DMA,
            pltpu.SemaphoreType.DMA,
        ],
    )
    def sc_embed_sum(x_hbm, table_hbm, out_hbm, idxc, rows, outc, tbl,
                     sem0, sem1, sem2, sem3):
        wid = lax.axis_index("s") * NC + lax.axis_index("c")
        base = wid * b_per_w
        zero = jnp.zeros((LANES,), jnp.float32)
        sems = (sem0, sem1, sem2, sem3)

        @pl.when(lax.axis_index("s") == 0)
        def _():
            pltpu.sync_copy(table_hbm, tbl)
        plsc.subcore_barrier()

        def issue(r, buf, sem):
            pltpu.async_copy(
                tbl.at[idxc.at[r, pl.ds(0, L_A)]],
                rows.at[buf, pl.ds(0, L_A)], sem)
            pltpu.async_copy(
                tbl.at[idxc.at[r, pl.ds(L_A, L_B)]],
                rows.at[buf, pl.ds(L_A, L_B)], sem)

        def wait(buf, sem):
            pltpu.make_async_copy(
                table_hbm.at[pl.ds(0, L)], rows.at[buf], sem).wait()

        def reduce_into(r, buf):
            def red(j10, accs):
                accs = list(accs)
                for u in range(10):
                    j = j10 * 20 + u * 2
                    for k2 in range(n_vregs // 2):
                        h0 = rows[buf, j, pl.ds(k2 * 2 * LANES, 2 * LANES)]
                        h1 = rows[buf, j + 1, pl.ds(k2 * 2 * LANES, 2 * LANES)]
                        a, b = plsc.unpack(h0 + h1,
                                           format=plsc.PackFormat.INTERLEAVED)
                        accs[2 * k2] = accs[2 * k2] + a
                        accs[2 * k2 + 1] = accs[2 * k2 + 1] + b
                return tuple(accs)

            accs = lax.fori_loop(0, L // 20, red, (zero,) * n_vregs)
            for k in range(n_vregs):
                outc[r, pl.ds(k * LANES, LANES)] = accs[k]

        def chunk_body(c, carry):
            row0 = base + c * R
            pltpu.sync_copy(x_hbm.at[pl.ds(row0, R)], idxc)
            for u in range(4):
                issue(u, u, sems[u])

            def quad_body(r4, carry2):
                r0 = 4 * r4
                for u in range(4):
                    r = r0 + u
                    wait(u, sems[u])
                    reduce_into(r, u)

                    @pl.when(r + 4 < R)
                    def _():
                        issue(r + 4, u, sems[u])
                return carry2

            lax.fori_loop(0, R // 4, quad_body, 0)
            pltpu.sync_copy(outc, out_hbm.at[pl.ds(row0, R)])
            return carry

        lax.fori_loop(0, n_chunks, chunk_body, 0)

    return sc_embed_sum


def _mlp_block(e_ref, w1_ref, b1_ref, w2t_ref, b2_ref, o_ref, *, inv_l):
    e = e_ref[...] * inv_l
    h = jnp.dot(e, w1_ref[...], preferred_element_type=jnp.float32) + b1_ref[...]
    h = jnp.maximum(h, 0.0)
    o_ref[...] = jnp.sum(h * w2t_ref[...], axis=1, keepdims=True) + b2_ref[...]


def _tc_mlp(esum, W1, b1, W2, b2, L):
    B, D = esum.shape
    H = W1.shape[1]
    BLK = 2048
    return pl.pallas_call(
        functools.partial(_mlp_block, inv_l=1.0 / L),
        grid=(B // BLK,),
        in_specs=[
            pl.BlockSpec((BLK, D), lambda i: (i, 0)),
            pl.BlockSpec((D, H), lambda i: (0, 0)),
            pl.BlockSpec((1, H), lambda i: (0, 0)),
            pl.BlockSpec((1, H), lambda i: (0, 0)),
            pl.BlockSpec((1, 1), lambda i: (0, 0)),
        ],
        out_specs=pl.BlockSpec((BLK, 1), lambda i: (i, 0)),
        out_shape=jax.ShapeDtypeStruct((B, 1), jnp.float32),
    )(esum, W1, b1.reshape(1, H), W2.reshape(1, H), b2.reshape(1, 1))


def kernel(x, table, W1, b1, W2, b2):
    B, L = x.shape
    D = table.shape[1]
    esum = _build_sc_embed_sum(B, L, D, table.shape[0])(
        x, table.astype(jnp.bfloat16))
    perm = np.concatenate(
        [np.concatenate([np.arange(g, g + 32, 2), np.arange(g + 1, g + 32, 2)])
         for g in range(0, D, 32)])
    out = _tc_mlp(esum, W1[perm, :], b1, W2, b2, L)
    return out[:, 0]


# speedup vs baseline: 1.3445x; 1.0050x over previous
import functools

import jax
import jax.numpy as jnp
import numpy as np
from jax import lax
from jax.experimental import pallas as pl
from jax.experimental.pallas import tpu as pltpu
from jax.experimental.pallas import tpu_sc as plsc


def _build_sc_embed_sum(B, L, D, V):
    info = plsc.get_sparse_core_info()
    NC, NS, LANES = info.num_cores, info.num_subcores, info.num_lanes
    NW = NC * NS
    b_per_w = B // NW
    R = 128
    n_chunks = b_per_w // R
    n_vregs = D // LANES
    L_A = 112
    L_B = L - L_A

    mesh = plsc.VectorSubcoreMesh(core_axis_name="c", subcore_axis_name="s")

    @functools.partial(
        pl.kernel,
        mesh=mesh,
        compiler_params=pltpu.CompilerParams(
            use_tc_tiling_on_sc=False, needs_layout_passes=False),
        out_type=jax.ShapeDtypeStruct((B, D), jnp.float32),
        scratch_types=[
            pltpu.VMEM((R, L), jnp.int32),
            pltpu.VMEM((4, L, D), jnp.bfloat16),
            pltpu.VMEM((R, D), jnp.float32),
            pltpu.VMEM_SHARED((V, D), jnp.bfloat16),
            pltpu.SemaphoreType.DMA,
            pltpu.SemaphoreType.DMA,
            pltpu.SemaphoreType.DMA,
            pltpu.SemaphoreType.DMA,
            pltpu.SemaphoreType.DMA,
            pltpu.SemaphoreType.DMA,
            pltpu.SemaphoreType.DMA,
            pltpu.SemaphoreType.DMA,
        ],
    )
    def sc_embed_sum(x_hbm, table_hbm, out_hbm, idxc, rows, outc, tbl,
                     sem0, sem1, sem2, sem3, sem4, sem5, sem6, sem7):
        wid = lax.axis_index("s") * NC + lax.axis_index("c")
        base = wid * b_per_w
        zero = jnp.zeros((LANES,), jnp.float32)
        sems = ((sem0, sem4), (sem1, sem5), (sem2, sem6), (sem3, sem7))

        @pl.when(lax.axis_index("s") == 0)
        def _():
            pltpu.sync_copy(table_hbm, tbl)
        plsc.subcore_barrier()

        def issue(r, buf, sem):
            pltpu.async_copy(
                tbl.at[idxc.at[r, pl.ds(0, L_A)]],
                rows.at[buf, pl.ds(0, L_A)], sem[0])
            pltpu.async_copy(
                table_hbm.at[idxc.at[r, pl.ds(L_A, L_B)]],
                rows.at[buf, pl.ds(L_A, L_B)], sem[1])

        def wait(buf, sem):
            pltpu.make_async_copy(
                table_hbm.at[pl.ds(0, L_A)],
                rows.at[buf, pl.ds(0, L_A)], sem[0]).wait()
            pltpu.make_async_copy(
                table_hbm.at[pl.ds(0, L_B)],
                rows.at[buf, pl.ds(L_A, L_B)], sem[1]).wait()

        def reduce_into(r, buf):
            def red(j10, accs):
                accs = list(accs)
                for u in range(10):
                    j = j10 * 20 + u * 2
                    for k2 in range(n_vregs // 2):
                        h0 = rows[buf, j, pl.ds(k2 * 2 * LANES, 2 * LANES)]
                        h1 = rows[buf, j + 1, pl.ds(k2 * 2 * LANES, 2 * LANES)]
                        a, b = plsc.unpack(h0 + h1,
                                           format=plsc.PackFormat.INTERLEAVED)
                        accs[2 * k2] = accs[2 * k2] + a
                        accs[2 * k2 + 1] = accs[2 * k2 + 1] + b
                return tuple(accs)

            accs = lax.fori_loop(0, L // 20, red, (zero,) * n_vregs)
            for k in range(n_vregs):
                outc[r, pl.ds(k * LANES, LANES)] = accs[k]

        def chunk_body(c, carry):
            row0 = base + c * R
            pltpu.sync_copy(x_hbm.at[pl.ds(row0, R)], idxc)
            for u in range(4):
                issue(u, u, sems[u])

            def quad_body(r4, carry2):
                r0 = 4 * r4
                for u in range(4):
                    r = r0 + u
                    wait(u, sems[u])
                    reduce_into(r, u)

                    @pl.when(r + 4 < R)
                    def _():
                        issue(r + 4, u, sems[u])
                return carry2

            lax.fori_loop(0, R // 4, quad_body, 0)
            pltpu.sync_copy(outc, out_hbm.at[pl.ds(row0, R)])
            return carry

        lax.fori_loop(0, n_chunks, chunk_body, 0)

    return sc_embed_sum


def _mlp_block(e_ref, w1_ref, b1_ref, w2t_ref, b2_ref, o_ref, *, inv_l):
    e = e_ref[...] * inv_l
    h = jnp.dot(e, w1_ref[...], preferred_element_type=jnp.float32) + b1_ref[...]
    h = jnp.maximum(h, 0.0)
    o_ref[...] = jnp.sum(h * w2t_ref[...], axis=1, keepdims=True) + b2_ref[...]


def _tc_mlp(esum, W1, b1, W2, b2, L):
    B, D = esum.shape
    H = W1.shape[1]
    BLK = 2048
    return pl.pallas_call(
        functools.partial(_mlp_block, inv_l=1.0 / L),
        grid=(B // BLK,),
        in_specs=[
            pl.BlockSpec((BLK, D), lambda i: (i, 0)),
            pl.BlockSpec((D, H), lambda i: (0, 0)),
            pl.BlockSpec((1, H), lambda i: (0, 0)),
            pl.BlockSpec((1, H), lambda i: (0, 0)),
            pl.BlockSpec((1, 1), lambda i: (0, 0)),
        ],
        out_specs=pl.BlockSpec((BLK, 1), lambda i: (i, 0)),
        out_shape=jax.ShapeDtypeStruct((B, 1), jnp.float32),
    )(esum, W1, b1.reshape(1, H), W2.reshape(1, H), b2.reshape(1, 1))


def kernel(x, table, W1, b1, W2, b2):
    B, L = x.shape
    D = table.shape[1]
    esum = _build_sc_embed_sum(B, L, D, table.shape[0])(
        x, table.astype(jnp.bfloat16))
    perm = np.concatenate(
        [np.concatenate([np.arange(g, g + 32, 2), np.arange(g + 1, g + 32, 2)])
         for g in range(0, D, 32)])
    out = _tc_mlp(esum, W1[perm, :], b1, W2, b2, L)
    return out[:, 0]
